# baseline (device time: 75340 ns/iter reference)
import jax
import jax.numpy as jnp
from jax import lax
from jax.experimental import pallas as pl
from jax.experimental.pallas import tpu as pltpu

N_DEV = 8
B, SQ, DM = 2, 128, 512
HQ_TOT, HQ, DH = 32, 4, 64
SKV_SHARD = 128
N_CHUNKS = 2
ROWS = SQ // N_DEV


def kernel(x, Wq, K_ext, V_ext, Wo):
    def body(x_ref, wq_ref, k_ref, v_ref, wo_ref, out_ref,
             kbuf, vbuf, partial_ref, rs_buf,
             kv_send_sems, kv_recv_sems,
             rs_send_sems, rs_recv_sems,
             ag_send_sems, ag_recv_sems):
        p = lax.axis_index("i")

        bsem = pltpu.get_barrier_semaphore()
        for off in range(1, N_DEV):
            pl.semaphore_signal(
                bsem, inc=1,
                device_id=((p + off) % N_DEV,),
                device_id_type=pl.DeviceIdType.MESH,
            )
        pl.semaphore_wait(bsem, N_DEV - 1)

        for j in range(N_CHUNKS):
            @pl.when(p == j)
            def _(j=j):
                kbuf[j] = k_ref[:, :, 4 * j:4 * j + HQ, :]
                vbuf[j] = v_ref[:, :, 4 * j:4 * j + HQ, :]
                sends = []
                for d in range(N_DEV):
                    if d == j:
                        continue
                    for t, (src_full, buf) in enumerate(
                            [(k_ref, kbuf), (v_ref, vbuf)]):
                        r = pltpu.make_async_remote_copy(
                            src_ref=src_full.at[:, :, pl.ds(4 * d, HQ), :],
                            dst_ref=buf.at[j],
                            send_sem=kv_send_sems.at[t, d],
                            recv_sem=kv_recv_sems.at[t, j],
                            device_id=(d,),
                            device_id_type=pl.DeviceIdType.MESH,
                        )
                        r.start()
                        sends.append(r)
                for r in sends:
                    r.wait_send()

        qs = []
        for b in range(B):
            qs.append(jnp.dot(x_ref[b], wq_ref[...],
                              preferred_element_type=jnp.float32))

        for j in range(N_CHUNKS):
            @pl.when(p != j)
            def _(j=j):
                for t, (src_full, buf) in enumerate(
                        [(k_ref, kbuf), (v_ref, vbuf)]):
                    r = pltpu.make_async_remote_copy(
                        src_ref=src_full.at[:, :, pl.ds(0, HQ), :],
                        dst_ref=buf.at[j],
                        send_sem=kv_send_sems.at[t, 0],
                        recv_sem=kv_recv_sems.at[t, j],
                        device_id=(j,),
                        device_id_type=pl.DeviceIdType.MESH,
                    )
                    r.wait_recv()

        skv = N_CHUNKS * SKV_SHARD
        qi = lax.broadcasted_iota(jnp.int32, (SQ, skv), 0)
        ki = lax.broadcasted_iota(jnp.int32, (SQ, skv), 1)
        mask = jnp.abs(qi - ki) <= 128

        for b in range(B):
            cols = []
            for h in range(HQ):
                qbh = qs[b][:, DH * h:DH * h + DH]
                kbh = jnp.concatenate(
                    [kbuf[0, b, :, h, :], kbuf[1, b, :, h, :]], axis=0)
                vbh = jnp.concatenate(
                    [vbuf[0, b, :, h, :], vbuf[1, b, :, h, :]], axis=0)
                s = lax.dot_general(
                    qbh, kbh, (((1,), (1,)), ((), ())),
                    preferred_element_type=jnp.float32) * 0.125
                s = jnp.where(mask, s, -1e9)
                m = jnp.max(s, axis=-1, keepdims=True)
                w = jnp.exp(s - m)
                w = w / jnp.sum(w, axis=-1, keepdims=True)
                cols.append(jnp.dot(w, vbh,
                                    preferred_element_type=jnp.float32))
            ctx_b = jnp.concatenate(cols, axis=1)
            partial_ref[b] = jnp.dot(ctx_b, wo_ref[...],
                                     preferred_element_type=jnp.float32)

        rs_sends = []
        for off in range(1, N_DEV):
            d = (p + off) % N_DEV
            r = pltpu.make_async_remote_copy(
                src_ref=partial_ref.at[:, pl.ds(ROWS * d, ROWS), :],
                dst_ref=rs_buf.at[p],
                send_sem=rs_send_sems.at[off - 1],
                recv_sem=rs_recv_sems.at[p],
                device_id=(d,),
                device_id_type=pl.DeviceIdType.MESH,
            )
            r.start()
            rs_sends.append(r)

        acc = partial_ref[:, pl.ds(ROWS * p, ROWS), :]
        for off in range(1, N_DEV):
            src = (p + off) % N_DEV
            r = pltpu.make_async_remote_copy(
                src_ref=partial_ref.at[:, pl.ds(0, ROWS), :],
                dst_ref=rs_buf.at[src],
                send_sem=rs_send_sems.at[0],
                recv_sem=rs_recv_sems.at[src],
                device_id=(src,),
                device_id_type=pl.DeviceIdType.MESH,
            )
            r.wait_recv()
            acc = acc + jnp.squeeze(rs_buf[pl.ds(src, 1)], axis=0)
        out_ref[:, pl.ds(ROWS * p, ROWS), :] = acc
        for r in rs_sends:
            r.wait_send()

        ag_sends = []
        for off in range(1, N_DEV):
            d = (p + off) % N_DEV
            r = pltpu.make_async_remote_copy(
                src_ref=out_ref.at[:, pl.ds(ROWS * p, ROWS), :],
                dst_ref=out_ref.at[:, pl.ds(ROWS * p, ROWS), :],
                send_sem=ag_send_sems.at[off - 1],
                recv_sem=ag_recv_sems.at[p],
                device_id=(d,),
                device_id_type=pl.DeviceIdType.MESH,
            )
            r.start()
            ag_sends.append(r)
        for off in range(1, N_DEV):
            src = (p + off) % N_DEV
            r = pltpu.make_async_remote_copy(
                src_ref=out_ref.at[:, pl.ds(0, ROWS), :],
                dst_ref=out_ref.at[:, pl.ds(ROWS * src, ROWS), :],
                send_sem=ag_send_sems.at[0],
                recv_sem=ag_recv_sems.at[src],
                device_id=(src,),
                device_id_type=pl.DeviceIdType.MESH,
            )
            r.wait_recv()
        for r in ag_sends:
            r.wait_send()

    return pl.pallas_call(
        body,
        out_shape=jax.ShapeDtypeStruct((B, SQ, DM), jnp.float32),
        in_specs=[pl.BlockSpec(memory_space=pltpu.VMEM)] * 5,
        out_specs=pl.BlockSpec(memory_space=pltpu.VMEM),
        scratch_shapes=[
            pltpu.VMEM((N_CHUNKS, B, SKV_SHARD, HQ, DH), jnp.float32),
            pltpu.VMEM((N_CHUNKS, B, SKV_SHARD, HQ, DH), jnp.float32),
            pltpu.VMEM((B, SQ, DM), jnp.float32),
            pltpu.VMEM((N_DEV, B, ROWS, DM), jnp.float32),
            pltpu.SemaphoreType.DMA((2, N_DEV)),
            pltpu.SemaphoreType.DMA((2, N_CHUNKS)),
            pltpu.SemaphoreType.DMA((N_DEV - 1,)),
            pltpu.SemaphoreType.DMA((N_DEV,)),
            pltpu.SemaphoreType.DMA((N_DEV - 1,)),
            pltpu.SemaphoreType.DMA((N_DEV,)),
        ],
        compiler_params=pltpu.CompilerParams(collective_id=0),
    )(x, Wq, K_ext, V_ext, Wo)


# device time: 51267 ns/iter; 1.4696x vs baseline; 1.4696x over previous
import jax
import jax.numpy as jnp
from jax import lax
from jax.experimental import pallas as pl
from jax.experimental.pallas import tpu as pltpu

N_DEV = 8
B, SQ, DM = 2, 128, 512
HQ, DH = 4, 64
SKV_SHARD = 128
N_CHUNKS = 2
ROWS = SQ // N_DEV
BF16 = jnp.bfloat16


def kernel(x, Wq, K_ext, V_ext, Wo):
    def body(x_ref, wq_ref, k_ref, v_ref, wo_ref, out_ref,
             kv_send_buf, kvbuf, partial_ref, rs_buf, red_ref, ag_buf,
             kv_send_sems, kv_recv_sems,
             rs_send_sems, rs_recv_sems,
             ag_send_sems, ag_recv_sems):
        p = lax.axis_index("i")

        def kv_send_rdma(j, d):
            return pltpu.make_async_remote_copy(
                src_ref=kv_send_buf.at[d],
                dst_ref=kvbuf.at[j],
                send_sem=kv_send_sems.at[d],
                recv_sem=kv_recv_sems.at[j],
                device_id=(d,),
                device_id_type=pl.DeviceIdType.MESH,
            )

        bsem = pltpu.get_barrier_semaphore()
        for off in range(1, N_DEV):
            pl.semaphore_signal(
                bsem, inc=1,
                device_id=((p + off) % N_DEV,),
                device_id_type=pl.DeviceIdType.MESH,
            )
        pl.semaphore_wait(bsem, N_DEV - 1)

        for j in range(N_CHUNKS):
            @pl.when(p == j)
            def _(j=j):
                kvbuf[j, 0] = k_ref[:, :, 4 * j:4 * j + HQ, :].astype(BF16)
                kvbuf[j, 1] = v_ref[:, :, 4 * j:4 * j + HQ, :].astype(BF16)
                for d in range(N_DEV):
                    if d == j:
                        continue
                    kv_send_buf[d, 0] = (
                        k_ref[:, :, 4 * d:4 * d + HQ, :].astype(BF16))
                    kv_send_buf[d, 1] = (
                        v_ref[:, :, 4 * d:4 * d + HQ, :].astype(BF16))
                    kv_send_rdma(j, d).start()

        qs = []
        for b in range(B):
            qs.append(jnp.dot(x_ref[b], wq_ref[...],
                              preferred_element_type=jnp.float32)
                      .astype(BF16))

        for j in range(N_CHUNKS):
            @pl.when(p != j)
            def _(j=j):
                kv_send_rdma(j, 0).wait_recv()

        skv = N_CHUNKS * SKV_SHARD
        qi = lax.broadcasted_iota(jnp.int32, (SQ, skv), 0)
        ki = lax.broadcasted_iota(jnp.int32, (SQ, skv), 1)
        mask = jnp.abs(qi - ki) <= 128

        for b in range(B):
            cols = []
            for h in range(HQ):
                qbh = qs[b][:, DH * h:DH * h + DH]
                kbh = jnp.concatenate(
                    [kvbuf[0, 0, b, :, h, :], kvbuf[1, 0, b, :, h, :]],
                    axis=0)
                vbh = jnp.concatenate(
                    [kvbuf[0, 1, b, :, h, :], kvbuf[1, 1, b, :, h, :]],
                    axis=0)
                s = lax.dot_general(
                    qbh, kbh, (((1,), (1,)), ((), ())),
                    preferred_element_type=jnp.float32) * 0.125
                s = jnp.where(mask, s, -1e9)
                m = jnp.max(s, axis=-1, keepdims=True)
                w = jnp.exp(s - m)
                w = (w / jnp.sum(w, axis=-1, keepdims=True)).astype(BF16)
                cols.append(jnp.dot(w, vbh,
                                    preferred_element_type=jnp.float32))
            ctx_b = jnp.concatenate(cols, axis=1).astype(BF16)
            partial_ref[b] = jnp.dot(
                ctx_b, wo_ref[...].astype(BF16),
                preferred_element_type=jnp.float32).astype(BF16)

        rs_sends = []
        for off in range(1, N_DEV):
            d = (p + off) % N_DEV
            r = pltpu.make_async_remote_copy(
                src_ref=partial_ref.at[:, pl.ds(ROWS * d, ROWS), :],
                dst_ref=rs_buf.at[p],
                send_sem=rs_send_sems.at[off - 1],
                recv_sem=rs_recv_sems.at[p],
                device_id=(d,),
                device_id_type=pl.DeviceIdType.MESH,
            )
            r.start()
            rs_sends.append(r)

        acc = partial_ref[:, pl.ds(ROWS * p, ROWS), :].astype(jnp.float32)
        for off in range(1, N_DEV):
            src = (p + off) % N_DEV
            r = pltpu.make_async_remote_copy(
                src_ref=partial_ref.at[:, pl.ds(0, ROWS), :],
                dst_ref=rs_buf.at[src],
                send_sem=rs_send_sems.at[0],
                recv_sem=rs_recv_sems.at[src],
                device_id=(src,),
                device_id_type=pl.DeviceIdType.MESH,
            )
            r.wait_recv()
            acc = acc + jnp.squeeze(
                rs_buf[pl.ds(src, 1)], axis=0).astype(jnp.float32)
        out_ref[:, pl.ds(ROWS * p, ROWS), :] = acc
        red_ref[...] = acc.astype(BF16)

        ag_sends = []
        for off in range(1, N_DEV):
            d = (p + off) % N_DEV
            r = pltpu.make_async_remote_copy(
                src_ref=red_ref,
                dst_ref=ag_buf.at[p],
                send_sem=ag_send_sems.at[off - 1],
                recv_sem=ag_recv_sems.at[p],
                device_id=(d,),
                device_id_type=pl.DeviceIdType.MESH,
            )
            r.start()
            ag_sends.append(r)
        for off in range(1, N_DEV):
            src = (p + off) % N_DEV
            r = pltpu.make_async_remote_copy(
                src_ref=red_ref,
                dst_ref=ag_buf.at[src],
                send_sem=ag_send_sems.at[0],
                recv_sem=ag_recv_sems.at[src],
                device_id=(src,),
                device_id_type=pl.DeviceIdType.MESH,
            )
            r.wait_recv()
            out_ref[:, pl.ds(ROWS * src, ROWS), :] = jnp.squeeze(
                ag_buf[pl.ds(src, 1)], axis=0).astype(jnp.float32)

        for r in rs_sends + ag_sends:
            r.wait_send()
        for j in range(N_CHUNKS):
            @pl.when(p == j)
            def _(j=j):
                for d in range(N_DEV):
                    if d != j:
                        kv_send_rdma(j, d).wait_send()

    return pl.pallas_call(
        body,
        out_shape=jax.ShapeDtypeStruct((B, SQ, DM), jnp.float32),
        in_specs=[pl.BlockSpec(memory_space=pltpu.VMEM)] * 5,
        out_specs=pl.BlockSpec(memory_space=pltpu.VMEM),
        scratch_shapes=[
            pltpu.VMEM((N_DEV, 2, B, SKV_SHARD, HQ, DH), BF16),
            pltpu.VMEM((N_CHUNKS, 2, B, SKV_SHARD, HQ, DH), BF16),
            pltpu.VMEM((B, SQ, DM), BF16),
            pltpu.VMEM((N_DEV, B, ROWS, DM), BF16),
            pltpu.VMEM((B, ROWS, DM), BF16),
            pltpu.VMEM((N_DEV, B, ROWS, DM), BF16),
            pltpu.SemaphoreType.DMA((N_DEV,)),
            pltpu.SemaphoreType.DMA((N_CHUNKS,)),
            pltpu.SemaphoreType.DMA((N_DEV - 1,)),
            pltpu.SemaphoreType.DMA((N_DEV,)),
            pltpu.SemaphoreType.DMA((N_DEV - 1,)),
            pltpu.SemaphoreType.DMA((N_DEV,)),
        ],
        compiler_params=pltpu.CompilerParams(collective_id=0),
    )(x, Wq, K_ext, V_ext, Wo)


# device time: 25689 ns/iter; 2.9328x vs baseline; 1.9957x over previous
import os

import jax
import jax.numpy as jnp
from jax import lax
from jax.experimental import pallas as pl
from jax.experimental.pallas import tpu as pltpu

_VAR = os.environ.get("KVAR", "")
SKIP_SCATTER = "S" in _VAR
SKIP_ATTN = "T" in _VAR
SKIP_AR = "R" in _VAR

N_DEV = 8
B, SQ, DM = 2, 128, 512
HQ, DH = 4, 64
SKV_SHARD = 128
N_CHUNKS = 2
ROWS = SQ // N_DEV
BF16 = jnp.bfloat16


def kernel(x, Wq, K_ext, V_ext, Wo):
    def body(x_ref, wq_ref, k_ref, v_ref, wo_ref, out_ref,
             kv_send_buf, kvbuf, partial_ref, rs_buf, red_ref, ag_buf,
             kv_send_sems, kv_recv_sems,
             rs_send_sems, rs_recv_sems,
             ag_send_sems, ag_recv_sems):
        p = lax.axis_index("i")

        def kv_send_rdma(j, d):
            return pltpu.make_async_remote_copy(
                src_ref=kv_send_buf.at[d],
                dst_ref=kvbuf.at[j],
                send_sem=kv_send_sems.at[d],
                recv_sem=kv_recv_sems.at[j],
                device_id=(d,),
                device_id_type=pl.DeviceIdType.MESH,
            )

        bsem = pltpu.get_barrier_semaphore()
        for off in range(1, N_DEV):
            pl.semaphore_signal(
                bsem, inc=1,
                device_id=((p + off) % N_DEV,),
                device_id_type=pl.DeviceIdType.MESH,
            )
        pl.semaphore_wait(bsem, N_DEV - 1)

        for j in range(N_CHUNKS) if not SKIP_SCATTER else []:
            @pl.when(p == j)
            def _(j=j):
                kvbuf[j, 0] = k_ref[:, :, 4 * j:4 * j + HQ, :].astype(BF16)
                kvbuf[j, 1] = v_ref[:, :, 4 * j:4 * j + HQ, :].astype(BF16)
                for d in range(N_DEV):
                    if d == j:
                        continue
                    kv_send_buf[d, 0] = (
                        k_ref[:, :, 4 * d:4 * d + HQ, :].astype(BF16))
                    kv_send_buf[d, 1] = (
                        v_ref[:, :, 4 * d:4 * d + HQ, :].astype(BF16))
                    kv_send_rdma(j, d).start()

        qs = []
        for b in range(B):
            qs.append(jnp.dot(x_ref[b], wq_ref[...],
                              preferred_element_type=jnp.float32)
                      .astype(BF16))

        for j in range(N_CHUNKS) if not SKIP_SCATTER else []:
            @pl.when(p != j)
            def _(j=j):
                kv_send_rdma(j, 0).wait_recv()

        skv = N_CHUNKS * SKV_SHARD
        qi = lax.broadcasted_iota(jnp.int32, (SQ, skv), 0)
        ki = lax.broadcasted_iota(jnp.int32, (SQ, skv), 1)
        mask = jnp.abs(qi - ki) <= 128

        for b in range(B) if not SKIP_ATTN else []:
            cols = []
            for h in range(HQ):
                qbh = qs[b][:, DH * h:DH * h + DH]
                kbh = jnp.concatenate(
                    [kvbuf[0, 0, b, :, h, :], kvbuf[1, 0, b, :, h, :]],
                    axis=0)
                vbh = jnp.concatenate(
                    [kvbuf[0, 1, b, :, h, :], kvbuf[1, 1, b, :, h, :]],
                    axis=0)
                s = lax.dot_general(
                    qbh, kbh, (((1,), (1,)), ((), ())),
                    preferred_element_type=jnp.float32) * 0.125
                s = jnp.where(mask, s, -1e9)
                m = jnp.max(s, axis=-1, keepdims=True)
                w = jnp.exp(s - m)
                w = (w / jnp.sum(w, axis=-1, keepdims=True)).astype(BF16)
                cols.append(jnp.dot(w, vbh,
                                    preferred_element_type=jnp.float32))
            ctx_b = jnp.concatenate(cols, axis=1).astype(BF16)
            partial_ref[b] = jnp.dot(
                ctx_b, wo_ref[...].astype(BF16),
                preferred_element_type=jnp.float32).astype(BF16)

        for j in range(N_CHUNKS) if not SKIP_SCATTER else []:
            @pl.when(p == j)
            def _(j=j):
                for d in range(N_DEV):
                    if d != j:
                        kv_send_rdma(j, d).wait_send()

        if SKIP_AR:
            out_ref[...] = partial_ref[...].astype(jnp.float32)
            return

        rs_sends = []
        for off in range(1, N_DEV):
            d = (p + off) % N_DEV
            r = pltpu.make_async_remote_copy(
                src_ref=partial_ref.at[:, pl.ds(ROWS * d, ROWS), :],
                dst_ref=rs_buf.at[p],
                send_sem=rs_send_sems.at[off - 1],
                recv_sem=rs_recv_sems.at[p],
                device_id=(d,),
                device_id_type=pl.DeviceIdType.MESH,
            )
            r.start()
            rs_sends.append(r)

        acc = partial_ref[:, pl.ds(ROWS * p, ROWS), :].astype(jnp.float32)
        for off in range(1, N_DEV):
            src = (p + off) % N_DEV
            r = pltpu.make_async_remote_copy(
                src_ref=partial_ref.at[:, pl.ds(0, ROWS), :],
                dst_ref=rs_buf.at[src],
                send_sem=rs_send_sems.at[0],
                recv_sem=rs_recv_sems.at[src],
                device_id=(src,),
                device_id_type=pl.DeviceIdType.MESH,
            )
            r.wait_recv()
            acc = acc + jnp.squeeze(
                rs_buf[pl.ds(src, 1)], axis=0).astype(jnp.float32)
        out_ref[:, pl.ds(ROWS * p, ROWS), :] = acc
        red_ref[...] = acc.astype(BF16)

        ag_sends = []
        for off in range(1, N_DEV):
            d = (p + off) % N_DEV
            r = pltpu.make_async_remote_copy(
                src_ref=red_ref,
                dst_ref=ag_buf.at[p],
                send_sem=ag_send_sems.at[off - 1],
                recv_sem=ag_recv_sems.at[p],
                device_id=(d,),
                device_id_type=pl.DeviceIdType.MESH,
            )
            r.start()
            ag_sends.append(r)
        for off in range(1, N_DEV):
            src = (p + off) % N_DEV
            r = pltpu.make_async_remote_copy(
                src_ref=red_ref,
                dst_ref=ag_buf.at[src],
                send_sem=ag_send_sems.at[0],
                recv_sem=ag_recv_sems.at[src],
                device_id=(src,),
                device_id_type=pl.DeviceIdType.MESH,
            )
            r.wait_recv()
            out_ref[:, pl.ds(ROWS * src, ROWS), :] = jnp.squeeze(
                ag_buf[pl.ds(src, 1)], axis=0).astype(jnp.float32)

        for r in rs_sends + ag_sends:
            r.wait_send()

    return pl.pallas_call(
        body,
        out_shape=jax.ShapeDtypeStruct((B, SQ, DM), jnp.float32),
        in_specs=[pl.BlockSpec(memory_space=pltpu.VMEM)] * 5,
        out_specs=pl.BlockSpec(memory_space=pltpu.VMEM),
        scratch_shapes=[
            pltpu.VMEM((N_DEV, 2, B, SKV_SHARD, HQ, DH), BF16),
            pltpu.VMEM((N_CHUNKS, 2, B, SKV_SHARD, HQ, DH), BF16),
            pltpu.VMEM((B, SQ, DM), BF16),
            pltpu.VMEM((N_DEV, B, ROWS, DM), BF16),
            pltpu.VMEM((B, ROWS, DM), BF16),
            pltpu.VMEM((N_DEV, B, ROWS, DM), BF16),
            pltpu.SemaphoreType.DMA((N_DEV,)),
            pltpu.SemaphoreType.DMA((N_CHUNKS,)),
            pltpu.SemaphoreType.DMA((N_DEV - 1,)),
            pltpu.SemaphoreType.DMA((N_DEV,)),
            pltpu.SemaphoreType.DMA((N_DEV - 1,)),
            pltpu.SemaphoreType.DMA((N_DEV,)),
        ],
        compiler_params=pltpu.CompilerParams(collective_id=0),
    )(x, Wq, K_ext, V_ext, Wo)
